# baseline (speedup 1.0000x reference)
import functools

import jax
import jax.numpy as jnp
import numpy as np
from jax import lax
from jax.experimental import pallas as pl
from jax.experimental.pallas import tpu as pltpu
from jax.experimental.pallas import tpu_sc as plsc

_N = 5000
_NPAD = 5120
_NGROUP = 20
_NEGF = np.float32(-np.inf)
_SCORE_THRESH = np.float32(0.2)
_NMS_THRESH = np.float32(0.5)
_MAXK = 15

_mesh = plsc.VectorSubcoreMesh(core_axis_name="c", subcore_axis_name="s",
                               num_cores=1)

_f32 = np.float32
_i32 = np.int32


def _iota16():
    return lax.broadcasted_iota(_i32, (16,), 0)


@functools.partial(
    pl.kernel,
    out_type=[jax.ShapeDtypeStruct((128,), _f32),
              jax.ShapeDtypeStruct((32,), _f32),
              jax.ShapeDtypeStruct((32,), _i32)],
    mesh=_mesh,
    compiler_params=pltpu.CompilerParams(needs_layout_passes=False),
    scratch_types=[
        pltpu.VMEM((4 * _N,), _f32),
        pltpu.VMEM((_NPAD,), _f32),
        pltpu.VMEM((_NPAD,), _i32),
        pltpu.VMEM((_NPAD,), _f32),
        pltpu.VMEM((_NGROUP * 16,), _f32),
        pltpu.VMEM((16,), _f32),
        pltpu.VMEM((16,), _i32),
        pltpu.VMEM((16,), _f32),
        pltpu.VMEM((16,), _i32),
        pltpu.VMEM((128,), _f32),
        pltpu.VMEM((32,), _f32),
        pltpu.VMEM((32,), _i32),
        pltpu.VMEM_SHARED((16,), _f32),
        pltpu.VMEM_SHARED((16,), _i32),
        pltpu.SemaphoreType.DMA,
    ],
)
def _nms_sc(bx_h, sc_h, lb_h,
            obx_h, osc_h, olb_h,
            bx_v, sc_v, lb_v, eff, cm,
            sS_v, sI_v, hS_v, hI_v,
            obx_v, obs_v, obl_v,
            shS, shI, sem):
    sid = lax.axis_index("s")
    iota = _iota16()

    def stage_prep_stream(want_human):
        copies = [
            pltpu.async_copy(bx_h, bx_v, sem),
            pltpu.async_copy(sc_h, sc_v.at[pl.ds(0, _N)], sem),
            pltpu.async_copy(lb_h, lb_v.at[pl.ds(0, _N)], sem),
        ]
        for c in copies:
            c.wait()

        def maxg(g, mv):
            base = g * 160
            for t in range(10):
                mv = jnp.maximum(mv, bx_v[pl.ds(base + t * 16, 16)])
            return mv

        mv = lax.fori_loop(0, 125, maxg, jnp.full((16,), _NEGF, _f32))
        maxc = jnp.max(mv) + _f32(1.0)

        def prep_group(g, carry):
            base0 = g * 256
            acc = jnp.full((16,), _NEGF, _f32)
            for t in range(16):
                base = base0 + t * 16
                scc = sc_v[pl.ds(base, 16)]
                lbc = lb_v[pl.ds(base, 16)]
                inb = (base + iota) < _N
                valid = jnp.logical_and(scc >= _SCORE_THRESH, inb)
                ish = lbc == 1
                want = ish if want_human else jnp.logical_not(ish)
                e = jnp.where(jnp.logical_and(valid, want), scc, _NEGF)
                eff[pl.ds(base, 16)] = e
                acc = jnp.where(iota == t, jnp.max(e), acc)
            cm[pl.ds(g * 16, 16)] = acc
            return carry

        lax.fori_loop(0, _NGROUP, prep_group, _i32(0))

        def cond(st):
            return jnp.logical_and(st[0] < _MAXK, jnp.logical_not(st[1]))

        def body(st):
            count, done, kx1, ky1, kx2, ky2, kar, selS, selI = st

            mvv = jnp.full((16,), _NEGF, _f32)
            ivv = jnp.zeros((16,), _i32)
            for j in range(_NGROUP):
                v = cm[pl.ds(j * 16, 16)]
                gt = v > mvv
                mvv = jnp.where(gt, v, mvv)
                ivv = jnp.where(gt, j * 16 + iota, ivv)
            m = jnp.max(mvv)
            valid_m = m > _f32(-1e38)
            chunk = jnp.min(jnp.where(mvv == m, ivv, _i32(1 << 30)))
            cbase = chunk * 16
            ev = eff[pl.ds(cbase, 16)]
            lane = jnp.min(jnp.where(ev == m, iota, _i32(15)))
            i = cbase + lane
            lm = iota == lane

            p0 = i * 4
            pb = (p0 // 16) * 16
            l0 = p0 - pb
            vb = bx_v[pl.ds(pb, 16)]

            def extc(k):
                return jnp.sum(jnp.where(iota == l0 + k, vb, _f32(0.0)))

            lbl = jnp.sum(jnp.where(lm, lb_v[pl.ds(cbase, 16)], _i32(0)))
            lblf = lbl.astype(_f32)
            off = lblf * maxc
            cx1 = extc(0) + off
            cy1 = extc(1) + off
            cx2 = extc(2) + off
            cy2 = extc(3) + off
            carea = (cx2 - cx1) * (cy2 - cy1)

            ltx = jnp.maximum(kx1, cx1)
            lty = jnp.maximum(ky1, cy1)
            rbx = jnp.minimum(kx2, cx2)
            rby = jnp.minimum(ky2, cy2)
            w = jnp.maximum(rbx - ltx, _f32(0.0))
            h = jnp.maximum(rby - lty, _f32(0.0))
            inter = w * h
            union = kar + carea - inter
            iou = inter / jnp.maximum(union, _f32(1e-9))
            supp = jnp.any(jnp.logical_and(iou > _NMS_THRESH, iota < count))
            keep = jnp.logical_and(jnp.logical_not(supp), valid_m)

            sel = jnp.logical_and(iota == count, keep)
            kx1 = jnp.where(sel, cx1, kx1)
            ky1 = jnp.where(sel, cy1, ky1)
            kx2 = jnp.where(sel, cx2, kx2)
            ky2 = jnp.where(sel, cy2, ky2)
            kar = jnp.where(sel, carea, kar)
            selS = jnp.where(sel, m, selS)
            selI = jnp.where(sel, i, selI)
            count = count + keep.astype(_i32)

            ev2 = jnp.where(lm, _NEGF, ev)
            eff[pl.ds(cbase, 16)] = ev2
            newmax = jnp.max(ev2)
            cb = (chunk // 16) * 16
            cv = cm[pl.ds(cb, 16)]
            cm[pl.ds(cb, 16)] = jnp.where(iota == chunk - cb, newmax, cv)
            done = jnp.logical_not(valid_m)
            return (count, done, kx1, ky1, kx2, ky2, kar, selS, selI)

        z16 = jnp.zeros((16,), _f32)
        st = lax.while_loop(
            cond, body,
            (_i32(0), False, z16, z16, z16, z16, z16,
             jnp.full((16,), _NEGF, _f32), jnp.zeros((16,), _i32)))
        sS_v[pl.ds(0, 16)] = st[7]
        sI_v[pl.ds(0, 16)] = st[8]

    @pl.when(sid == 0)
    def _object_stream():
        stage_prep_stream(want_human=False)

    @pl.when(sid == 1)
    def _human_stream():
        stage_prep_stream(want_human=True)
        pltpu.sync_copy(sS_v, shS)
        pltpu.sync_copy(sI_v, shI)

    plsc.subcore_barrier()

    @pl.when(sid == 0)
    def _merge():
        pltpu.sync_copy(shS, hS_v)
        pltpu.sync_copy(shI, hI_v)
        hS = hS_v[pl.ds(0, 16)]
        hI = hI_v[pl.ds(0, 16)]
        oS = sS_v[pl.ds(0, 16)]
        oI = sI_v[pl.ds(0, 16)]

        z16 = jnp.zeros((16,), _f32)
        for j in range(2):
            obs_v[pl.ds(j * 16, 16)] = z16
            obl_v[pl.ds(j * 16, 16)] = jnp.full((16,), -1, _i32)
        for j in range(8):
            obx_v[pl.ds(j * 16, 16)] = z16

        def fext(vec, p):
            return jnp.sum(jnp.where(iota == p, vec, _f32(0.0)))

        def iext(vec, p):
            return jnp.sum(jnp.where(iota == p, vec, _i32(0)))

        def mbody(k, c):
            a, b = c
            ha = fext(hS, a)
            hi_ = iext(hI, a)
            oa = fext(oS, b)
            oi_ = iext(oI, b)
            take_h = jnp.logical_or(
                ha > oa, jnp.logical_and(ha == oa, hi_ < oi_))
            any_ = jnp.maximum(ha, oa) > _f32(-1e38)
            i = jnp.where(take_h, hi_, oi_)
            s = jnp.where(take_h, ha, oa)
            p0 = i * 4
            pb = (p0 // 16) * 16
            l0 = p0 - pb
            vb = bx_v[pl.ds(pb, 16)]
            ibase = (i // 16) * 16
            ilm = iota == i - ibase
            ssc = jnp.where(any_, s, _f32(0.0))
            ilbl = jnp.sum(jnp.where(ilm, lb_v[pl.ds(ibase, 16)], _i32(0)))
            slb = jnp.where(any_, ilbl, _i32(-1))
            q0 = k * 4
            qb = (q0 // 16) * 16
            ql = q0 - qb
            rowv = jnp.zeros((16,), _f32)
            for d in range(4):
                cval = jnp.sum(jnp.where(iota == l0 + d, vb, _f32(0.0)))
                cval = jnp.where(any_, cval, _f32(0.0))
                rowv = jnp.where(iota == ql + d, cval, rowv)
            cur = obx_v[pl.ds(qb, 16)]
            inrow = jnp.logical_and(iota >= ql, iota < ql + 4)
            obx_v[pl.ds(qb, 16)] = jnp.where(inrow, rowv, cur)
            kb = (k // 16) * 16
            kl = k - kb
            km = iota == kl
            obs_v[pl.ds(kb, 16)] = jnp.where(km, ssc, obs_v[pl.ds(kb, 16)])
            obl_v[pl.ds(kb, 16)] = jnp.where(km, slb, obl_v[pl.ds(kb, 16)])
            taken = any_.astype(_i32)
            a = a + jnp.where(take_h, taken, 0)
            b = b + jnp.where(take_h, 0, taken)
            return (a, b)

        lax.fori_loop(0, 30, mbody, (_i32(0), _i32(0)))

        outs = [
            pltpu.async_copy(obx_v, obx_h, sem),
            pltpu.async_copy(obs_v, osc_h, sem),
            pltpu.async_copy(obl_v, olb_h, sem),
        ]
        for c in outs:
            c.wait()


def kernel(boxes, scores, labels):
    b128, osc, olb = _nms_sc(boxes.reshape(-1), scores, labels)
    return b128.reshape(32, 4)[:30], osc[:30], olb[:30]


# speedup vs baseline: 1.0242x; 1.0242x over previous
import functools

import jax
import jax.numpy as jnp
import numpy as np
from jax import lax
from jax.experimental import pallas as pl
from jax.experimental.pallas import tpu as pltpu
from jax.experimental.pallas import tpu_sc as plsc

_N = 5000
_NPAD = 5120
_NGROUP = 20
_QW = 5000
_NEGF = np.float32(-np.inf)
_SCORE_THRESH = np.float32(0.2)
_NMS_THRESH = np.float32(0.5)
_MAXK = 15

_mesh = plsc.VectorSubcoreMesh(core_axis_name="c", subcore_axis_name="s",
                               num_cores=1)

_f32 = np.float32
_i32 = np.int32


def _iota16():
    return lax.broadcasted_iota(_i32, (16,), 0)


@functools.partial(
    pl.kernel,
    out_type=[jax.ShapeDtypeStruct((128,), _f32),
              jax.ShapeDtypeStruct((32,), _f32),
              jax.ShapeDtypeStruct((32,), _i32)],
    mesh=_mesh,
    compiler_params=pltpu.CompilerParams(needs_layout_passes=False),
    scratch_types=[
        pltpu.VMEM((4 * _N,), _f32),
        pltpu.VMEM((_NPAD,), _f32),
        pltpu.VMEM((_NPAD,), _i32),
        pltpu.VMEM((_NPAD,), _f32),
        pltpu.VMEM((_NGROUP * 16,), _f32),
        pltpu.VMEM((32,), _f32),
        pltpu.VMEM((5008,), _f32),
        pltpu.VMEM((16,), _f32),
        pltpu.VMEM((64,), _f32),
        pltpu.VMEM((16,), _f32),
        pltpu.VMEM((16,), _i32),
        pltpu.VMEM((16,), _f32),
        pltpu.VMEM((16,), _i32),
        pltpu.VMEM((128,), _f32),
        pltpu.VMEM((32,), _f32),
        pltpu.VMEM((32,), _i32),
        pltpu.VMEM_SHARED((64,), _f32),
        pltpu.VMEM_SHARED((16,), _f32),
        pltpu.VMEM_SHARED((16,), _i32),
        pltpu.SemaphoreType.DMA,
        pltpu.SemaphoreType.DMA,
    ],
)
def _nms_sc(bx_h, sc_h, lb_h,
            obx_h, osc_h, olb_h,
            bx_v, sc_v, lb_v, eff, cm, cm2,
            wq_v, wm_v, pm_v,
            sS_v, sI_v, hS_v, hI_v,
            obx_v, obs_v, obl_v,
            shMax, shS, shI, sem, sem_bx):
    sid = lax.axis_index("s")
    iota = _iota16()

    @pl.when(jnp.logical_and(sid >= 2, sid <= 5))
    def _coord_max():
        pltpu.sync_copy(bx_h.at[pl.ds((sid - 2) * _QW, _QW)],
                        wq_v.at[pl.ds(0, _QW)])

        def maxg(g, mv):
            base = g * 128
            for t in range(8):
                mv = jnp.maximum(mv, wq_v[pl.ds(base + t * 16, 16)])
            return mv

        mv = lax.fori_loop(0, 39, maxg, jnp.full((16,), _NEGF, _f32))
        tail = wq_v[pl.ds(4992, 16)]
        mv = jnp.maximum(mv, jnp.where(iota < 8, tail, _NEGF))
        wm_v[pl.ds(0, 16)] = mv
        pltpu.sync_copy(wm_v, shMax.at[pl.ds((sid - 2) * 16, 16)])

    def stage_prep(want_human):
        bx_copy = pltpu.async_copy(bx_h, bx_v, sem_bx)
        copies = [
            pltpu.async_copy(sc_h, sc_v.at[pl.ds(0, _N)], sem),
            pltpu.async_copy(lb_h, lb_v.at[pl.ds(0, _N)], sem),
        ]
        copies[0].wait()
        copies[1].wait()

        def prep_group(g, carry):
            base0 = g * 256
            acc = jnp.full((16,), _NEGF, _f32)
            for t in range(16):
                base = base0 + t * 16
                scc = sc_v[pl.ds(base, 16)]
                lbc = lb_v[pl.ds(base, 16)]
                inb = (base + iota) < _N
                valid = jnp.logical_and(scc >= _SCORE_THRESH, inb)
                ish = lbc == 1
                want = ish if want_human else jnp.logical_not(ish)
                e = jnp.where(jnp.logical_and(valid, want), scc, _NEGF)
                eff[pl.ds(base, 16)] = e
                acc = jnp.where(iota == t, jnp.max(e), acc)
            cm[pl.ds(g * 16, 16)] = acc
            return carry

        lax.fori_loop(0, _NGROUP, prep_group, _i32(0))

        acc2 = jnp.full((16,), _NEGF, _f32)
        for g in range(16):
            acc2 = jnp.where(iota == g, jnp.max(cm[pl.ds(g * 16, 16)]), acc2)
        cm2[pl.ds(0, 16)] = acc2
        acc2 = jnp.full((16,), _NEGF, _f32)
        for g in range(16, _NGROUP):
            acc2 = jnp.where(iota == g - 16,
                             jnp.max(cm[pl.ds(g * 16, 16)]), acc2)
        cm2[pl.ds(16, 16)] = acc2
        bx_copy.wait()

    def run_stream():
        pltpu.sync_copy(shMax, pm_v)
        mv = jnp.maximum(
            jnp.maximum(pm_v[pl.ds(0, 16)], pm_v[pl.ds(16, 16)]),
            jnp.maximum(pm_v[pl.ds(32, 16)], pm_v[pl.ds(48, 16)]))
        maxc = jnp.max(mv) + _f32(1.0)

        def cond(st):
            return jnp.logical_and(st[0] < _MAXK, jnp.logical_not(st[1]))

        def body(st):
            count, done, kx1, ky1, kx2, ky2, kar, selS, selI = st

            mvv = jnp.full((16,), _NEGF, _f32)
            ivv = jnp.zeros((16,), _i32)
            for j in range(2):
                v = cm2[pl.ds(j * 16, 16)]
                gt = v > mvv
                mvv = jnp.where(gt, v, mvv)
                ivv = jnp.where(gt, j * 16 + iota, ivv)
            m = jnp.max(mvv)
            valid_m = m > _f32(-1e38)
            g = jnp.min(jnp.where(mvv == m, ivv, _i32(1 << 30)))
            gb = g * 16
            cmv = cm[pl.ds(gb, 16)]
            chunk = gb + jnp.min(jnp.where(cmv == m, iota, _i32(15)))
            cbase = chunk * 16
            ev = eff[pl.ds(cbase, 16)]
            lane = jnp.min(jnp.where(ev == m, iota, _i32(15)))
            i = cbase + lane
            lm = iota == lane

            p0 = i * 4
            pb = (p0 // 16) * 16
            l0 = p0 - pb
            vb = bx_v[pl.ds(pb, 16)]

            def extc(k):
                return jnp.sum(jnp.where(iota == l0 + k, vb, _f32(0.0)))

            lbl = jnp.sum(jnp.where(lm, lb_v[pl.ds(cbase, 16)], _i32(0)))
            lblf = lbl.astype(_f32)
            off = lblf * maxc
            cx1 = extc(0) + off
            cy1 = extc(1) + off
            cx2 = extc(2) + off
            cy2 = extc(3) + off
            carea = (cx2 - cx1) * (cy2 - cy1)

            ltx = jnp.maximum(kx1, cx1)
            lty = jnp.maximum(ky1, cy1)
            rbx = jnp.minimum(kx2, cx2)
            rby = jnp.minimum(ky2, cy2)
            w = jnp.maximum(rbx - ltx, _f32(0.0))
            h = jnp.maximum(rby - lty, _f32(0.0))
            inter = w * h
            union = kar + carea - inter
            iou = inter / jnp.maximum(union, _f32(1e-9))
            supp = jnp.any(jnp.logical_and(iou > _NMS_THRESH, iota < count))
            keep = jnp.logical_and(jnp.logical_not(supp), valid_m)

            sel = jnp.logical_and(iota == count, keep)
            kx1 = jnp.where(sel, cx1, kx1)
            ky1 = jnp.where(sel, cy1, ky1)
            kx2 = jnp.where(sel, cx2, kx2)
            ky2 = jnp.where(sel, cy2, ky2)
            kar = jnp.where(sel, carea, kar)
            selS = jnp.where(sel, m, selS)
            selI = jnp.where(sel, i, selI)
            count = count + keep.astype(_i32)

            ev2 = jnp.where(lm, _NEGF, ev)
            eff[pl.ds(cbase, 16)] = ev2
            newmax = jnp.max(ev2)
            cmv2 = jnp.where(iota == chunk - gb, newmax, cmv)
            cm[pl.ds(gb, 16)] = cmv2
            newgmax = jnp.max(cmv2)
            g2b = (g // 16) * 16
            c2v = cm2[pl.ds(g2b, 16)]
            cm2[pl.ds(g2b, 16)] = jnp.where(iota == g - g2b, newgmax, c2v)
            done = jnp.logical_not(valid_m)
            return (count, done, kx1, ky1, kx2, ky2, kar, selS, selI)

        z16 = jnp.zeros((16,), _f32)
        st = lax.while_loop(
            cond, body,
            (_i32(0), False, z16, z16, z16, z16, z16,
             jnp.full((16,), _NEGF, _f32), jnp.zeros((16,), _i32)))
        sS_v[pl.ds(0, 16)] = st[7]
        sI_v[pl.ds(0, 16)] = st[8]

    @pl.when(sid == 0)
    def _object_prep():
        stage_prep(want_human=False)

    @pl.when(sid == 1)
    def _human_prep():
        stage_prep(want_human=True)

    plsc.subcore_barrier()

    @pl.when(sid == 0)
    def _object_stream():
        run_stream()

    @pl.when(sid == 1)
    def _human_stream():
        run_stream()
        pltpu.sync_copy(sS_v, shS)
        pltpu.sync_copy(sI_v, shI)

    plsc.subcore_barrier()

    @pl.when(sid == 0)
    def _merge():
        pltpu.sync_copy(shS, hS_v)
        pltpu.sync_copy(shI, hI_v)
        hS = hS_v[pl.ds(0, 16)]
        hI = hI_v[pl.ds(0, 16)]
        oS = sS_v[pl.ds(0, 16)]
        oI = sI_v[pl.ds(0, 16)]

        z16 = jnp.zeros((16,), _f32)
        for j in range(2):
            obs_v[pl.ds(j * 16, 16)] = z16
            obl_v[pl.ds(j * 16, 16)] = jnp.full((16,), -1, _i32)
        for j in range(8):
            obx_v[pl.ds(j * 16, 16)] = z16

        def fext(vec, p):
            return jnp.sum(jnp.where(iota == p, vec, _f32(0.0)))

        def iext(vec, p):
            return jnp.sum(jnp.where(iota == p, vec, _i32(0)))

        def mbody(k, c):
            a, b = c
            ha = fext(hS, a)
            hi_ = iext(hI, a)
            oa = fext(oS, b)
            oi_ = iext(oI, b)
            take_h = jnp.logical_or(
                ha > oa, jnp.logical_and(ha == oa, hi_ < oi_))
            any_ = jnp.maximum(ha, oa) > _f32(-1e38)
            i = jnp.where(take_h, hi_, oi_)
            s = jnp.where(take_h, ha, oa)
            p0 = i * 4
            pb = (p0 // 16) * 16
            l0 = p0 - pb
            vb = bx_v[pl.ds(pb, 16)]
            ibase = (i // 16) * 16
            ilm = iota == i - ibase
            ssc = jnp.where(any_, s, _f32(0.0))
            ilbl = jnp.sum(jnp.where(ilm, lb_v[pl.ds(ibase, 16)], _i32(0)))
            slb = jnp.where(any_, ilbl, _i32(-1))
            q0 = k * 4
            qb = (q0 // 16) * 16
            ql = q0 - qb
            rowv = jnp.zeros((16,), _f32)
            for d in range(4):
                cval = jnp.sum(jnp.where(iota == l0 + d, vb, _f32(0.0)))
                cval = jnp.where(any_, cval, _f32(0.0))
                rowv = jnp.where(iota == ql + d, cval, rowv)
            cur = obx_v[pl.ds(qb, 16)]
            inrow = jnp.logical_and(iota >= ql, iota < ql + 4)
            obx_v[pl.ds(qb, 16)] = jnp.where(inrow, rowv, cur)
            kb = (k // 16) * 16
            kl = k - kb
            km = iota == kl
            obs_v[pl.ds(kb, 16)] = jnp.where(km, ssc, obs_v[pl.ds(kb, 16)])
            obl_v[pl.ds(kb, 16)] = jnp.where(km, slb, obl_v[pl.ds(kb, 16)])
            taken = any_.astype(_i32)
            a = a + jnp.where(take_h, taken, 0)
            b = b + jnp.where(take_h, 0, taken)
            return (a, b)

        lax.fori_loop(0, 30, mbody, (_i32(0), _i32(0)))

        outs = [
            pltpu.async_copy(obx_v, obx_h, sem),
            pltpu.async_copy(obs_v, osc_h, sem),
            pltpu.async_copy(obl_v, olb_h, sem),
        ]
        for c in outs:
            c.wait()


def kernel(boxes, scores, labels):
    b128, osc, olb = _nms_sc(boxes.reshape(-1), scores, labels)
    return b128.reshape(32, 4)[:30], osc[:30], olb[:30]


# speedup vs baseline: 1.0645x; 1.0394x over previous
import functools

import jax
import jax.numpy as jnp
import numpy as np
from jax import lax
from jax.experimental import pallas as pl
from jax.experimental.pallas import tpu as pltpu
from jax.experimental.pallas import tpu_sc as plsc

_N = 5000
_NPAD = 5120
_NGROUP = 20
_NEGF = np.float32(-np.inf)
_SCORE_THRESH = np.float32(0.2)
_NMS_THRESH = np.float32(0.5)
_MAXK = 15

_mesh = plsc.VectorSubcoreMesh(core_axis_name="c", subcore_axis_name="s",
                               num_cores=1)

_f32 = np.float32
_i32 = np.int32


def _iota16():
    return lax.broadcasted_iota(_i32, (16,), 0)


@functools.partial(
    pl.kernel,
    out_type=[jax.ShapeDtypeStruct((32,), _f32)] * 5
    + [jax.ShapeDtypeStruct((32,), _i32)],
    mesh=_mesh,
    compiler_params=pltpu.CompilerParams(needs_layout_passes=False),
    scratch_types=[
        pltpu.VMEM((_NPAD,), _f32),
        pltpu.VMEM((_NPAD,), _f32),
        pltpu.VMEM((_NPAD,), _f32),
        pltpu.VMEM((_NPAD,), _f32),
        pltpu.VMEM((_NPAD,), _f32),
        pltpu.VMEM((_NPAD,), _i32),
        pltpu.VMEM((_NPAD,), _f32),
        pltpu.VMEM((_NGROUP * 16,), _f32),
        pltpu.VMEM((32,), _f32),
        pltpu.VMEM((_NPAD,), _f32),
        pltpu.VMEM((16,), _f32),
        pltpu.VMEM((32,), _f32),
        pltpu.VMEM((16,), _f32),
        pltpu.VMEM((16,), _i32),
        pltpu.VMEM((16,), _f32),
        pltpu.VMEM((16,), _i32),
        pltpu.VMEM((32,), _f32),
        pltpu.VMEM((32,), _f32),
        pltpu.VMEM((32,), _f32),
        pltpu.VMEM((32,), _f32),
        pltpu.VMEM((32,), _f32),
        pltpu.VMEM((32,), _i32),
        pltpu.VMEM_SHARED((32,), _f32),
        pltpu.VMEM_SHARED((16,), _f32),
        pltpu.VMEM_SHARED((16,), _i32),
        pltpu.SemaphoreType.DMA,
        pltpu.SemaphoreType.DMA,
    ],
)
def _nms_sc(x1_h, y1_h, x2_h, y2_h, sc_h, lb_h,
            ox1_h, oy1_h, ox2_h, oy2_h, osc_h, olb_h,
            x1_v, y1_v, x2_v, y2_v, sc_v, lb_v,
            eff, cm, cm2, wq_v, wm_v, pm_v,
            sS_v, sI_v, hS_v, hI_v,
            ob1_v, ob2_v, ob3_v, ob4_v, obs_v, obl_v,
            shMax, shS, shI, sem, sem_c):
    sid = lax.axis_index("s")
    iota = _iota16()

    @pl.when(jnp.logical_or(sid == 2, sid == 3))
    def _coord_max():
        @pl.when(sid == 2)
        def _():
            pltpu.sync_copy(x2_h, wq_v)

        @pl.when(sid == 3)
        def _():
            pltpu.sync_copy(y2_h, wq_v)

        def maxg(g, mv):
            base = g * 256
            for t in range(16):
                mv = jnp.maximum(mv, wq_v[pl.ds(base + t * 16, 16)])
            return mv

        mv = lax.fori_loop(0, _NGROUP, maxg, jnp.full((16,), _NEGF, _f32))
        wm_v[pl.ds(0, 16)] = mv
        pltpu.sync_copy(wm_v, shMax.at[pl.ds((sid - 2) * 16, 16)])

    def stage_prep(want_human):
        sl_copies = [
            pltpu.async_copy(sc_h, sc_v, sem),
            pltpu.async_copy(lb_h, lb_v, sem),
        ]
        co_copies = [
            pltpu.async_copy(x1_h, x1_v, sem_c),
            pltpu.async_copy(y1_h, y1_v, sem_c),
            pltpu.async_copy(x2_h, x2_v, sem_c),
            pltpu.async_copy(y2_h, y2_v, sem_c),
        ]
        for c in sl_copies:
            c.wait()

        def prep_group(g, carry):
            base0 = g * 256
            acc = jnp.full((16,), _NEGF, _f32)
            for t in range(16):
                base = base0 + t * 16
                scc = sc_v[pl.ds(base, 16)]
                lbc = lb_v[pl.ds(base, 16)]
                valid = scc >= _SCORE_THRESH
                ish = lbc == 1
                want = ish if want_human else jnp.logical_not(ish)
                e = jnp.where(jnp.logical_and(valid, want), scc, _NEGF)
                eff[pl.ds(base, 16)] = e
                acc = jnp.where(iota == t, jnp.max(e), acc)
            cm[pl.ds(g * 16, 16)] = acc
            return carry

        lax.fori_loop(0, _NGROUP, prep_group, _i32(0))

        acc2 = jnp.full((16,), _NEGF, _f32)
        for g in range(16):
            acc2 = jnp.where(iota == g, jnp.max(cm[pl.ds(g * 16, 16)]), acc2)
        cm2[pl.ds(0, 16)] = acc2
        acc2 = jnp.full((16,), _NEGF, _f32)
        for g in range(16, _NGROUP):
            acc2 = jnp.where(iota == g - 16,
                             jnp.max(cm[pl.ds(g * 16, 16)]), acc2)
        cm2[pl.ds(16, 16)] = acc2
        for c in co_copies:
            c.wait()

    def run_stream():
        pltpu.sync_copy(shMax, pm_v)
        mv = jnp.maximum(pm_v[pl.ds(0, 16)], pm_v[pl.ds(16, 16)])
        maxc = jnp.max(mv) + _f32(1.0)

        def cond(st):
            return jnp.logical_and(st[0] < _MAXK, jnp.logical_not(st[1]))

        def body(st):
            count, done, kx1, ky1, kx2, ky2, kar, selS, selI = st

            mvv = jnp.full((16,), _NEGF, _f32)
            ivv = jnp.zeros((16,), _i32)
            for j in range(2):
                v = cm2[pl.ds(j * 16, 16)]
                gt = v > mvv
                mvv = jnp.where(gt, v, mvv)
                ivv = jnp.where(gt, j * 16 + iota, ivv)
            m = jnp.max(mvv)
            valid_m = m > _f32(-1e38)
            g = jnp.min(jnp.where(mvv == m, ivv, _i32(1 << 30)))
            gb = g * 16
            cmv = cm[pl.ds(gb, 16)]
            chunk = gb + jnp.min(jnp.where(cmv == m, iota, _i32(15)))
            cbase = chunk * 16
            ev = eff[pl.ds(cbase, 16)]
            lane = jnp.min(jnp.where(ev == m, iota, _i32(15)))
            i = cbase + lane
            lm = iota == lane

            def extf(ref):
                return jnp.sum(jnp.where(lm, ref[pl.ds(cbase, 16)],
                                         _f32(0.0)))

            lbl = jnp.sum(jnp.where(lm, lb_v[pl.ds(cbase, 16)], _i32(0)))
            lblf = lbl.astype(_f32)
            off = lblf * maxc
            cx1 = extf(x1_v) + off
            cy1 = extf(y1_v) + off
            cx2 = extf(x2_v) + off
            cy2 = extf(y2_v) + off
            carea = (cx2 - cx1) * (cy2 - cy1)

            ltx = jnp.maximum(kx1, cx1)
            lty = jnp.maximum(ky1, cy1)
            rbx = jnp.minimum(kx2, cx2)
            rby = jnp.minimum(ky2, cy2)
            w = jnp.maximum(rbx - ltx, _f32(0.0))
            h = jnp.maximum(rby - lty, _f32(0.0))
            inter = w * h
            union = kar + carea - inter
            iou = inter / jnp.maximum(union, _f32(1e-9))
            supp = jnp.any(jnp.logical_and(iou > _NMS_THRESH, iota < count))
            keep = jnp.logical_and(jnp.logical_not(supp), valid_m)

            sel = jnp.logical_and(iota == count, keep)
            kx1 = jnp.where(sel, cx1, kx1)
            ky1 = jnp.where(sel, cy1, ky1)
            kx2 = jnp.where(sel, cx2, kx2)
            ky2 = jnp.where(sel, cy2, ky2)
            kar = jnp.where(sel, carea, kar)
            selS = jnp.where(sel, m, selS)
            selI = jnp.where(sel, i, selI)
            count = count + keep.astype(_i32)

            ev2 = jnp.where(lm, _NEGF, ev)
            eff[pl.ds(cbase, 16)] = ev2
            newmax = jnp.max(ev2)
            cmv2 = jnp.where(iota == chunk - gb, newmax, cmv)
            cm[pl.ds(gb, 16)] = cmv2
            newgmax = jnp.max(cmv2)
            g2b = (g // 16) * 16
            c2v = cm2[pl.ds(g2b, 16)]
            cm2[pl.ds(g2b, 16)] = jnp.where(iota == g - g2b, newgmax, c2v)
            done = jnp.logical_not(valid_m)
            return (count, done, kx1, ky1, kx2, ky2, kar, selS, selI)

        z16 = jnp.zeros((16,), _f32)
        st = lax.while_loop(
            cond, body,
            (_i32(0), False, z16, z16, z16, z16, z16,
             jnp.full((16,), _NEGF, _f32), jnp.zeros((16,), _i32)))
        sS_v[pl.ds(0, 16)] = st[7]
        sI_v[pl.ds(0, 16)] = st[8]

    @pl.when(sid == 0)
    def _object_prep():
        stage_prep(want_human=False)

    @pl.when(sid == 1)
    def _human_prep():
        stage_prep(want_human=True)

    plsc.subcore_barrier()

    @pl.when(sid == 0)
    def _object_stream():
        run_stream()

    @pl.when(sid == 1)
    def _human_stream():
        run_stream()
        pltpu.sync_copy(sS_v, shS)
        pltpu.sync_copy(sI_v, shI)

    plsc.subcore_barrier()

    @pl.when(sid == 0)
    def _merge():
        pltpu.sync_copy(shS, hS_v)
        pltpu.sync_copy(shI, hI_v)
        hS = hS_v[pl.ds(0, 16)]
        hI = hI_v[pl.ds(0, 16)]
        oS = sS_v[pl.ds(0, 16)]
        oI = sI_v[pl.ds(0, 16)]

        z16 = jnp.zeros((16,), _f32)
        for j in range(2):
            ob1_v[pl.ds(j * 16, 16)] = z16
            ob2_v[pl.ds(j * 16, 16)] = z16
            ob3_v[pl.ds(j * 16, 16)] = z16
            ob4_v[pl.ds(j * 16, 16)] = z16
            obs_v[pl.ds(j * 16, 16)] = z16
            obl_v[pl.ds(j * 16, 16)] = jnp.full((16,), -1, _i32)

        def fext(vec, p):
            return jnp.sum(jnp.where(iota == p, vec, _f32(0.0)))

        def iext(vec, p):
            return jnp.sum(jnp.where(iota == p, vec, _i32(0)))

        def mbody(k, c):
            a, b = c
            ha = fext(hS, a)
            hi_ = iext(hI, a)
            oa = fext(oS, b)
            oi_ = iext(oI, b)
            take_h = jnp.logical_or(
                ha > oa, jnp.logical_and(ha == oa, hi_ < oi_))
            any_ = jnp.maximum(ha, oa) > _f32(-1e38)
            i = jnp.where(take_h, hi_, oi_)
            s = jnp.where(take_h, ha, oa)
            ibase = (i // 16) * 16
            ilm = iota == i - ibase

            def gext(ref):
                return jnp.sum(jnp.where(ilm, ref[pl.ds(ibase, 16)],
                                         _f32(0.0)))

            bx1 = jnp.where(any_, gext(x1_v), _f32(0.0))
            by1 = jnp.where(any_, gext(y1_v), _f32(0.0))
            bx2 = jnp.where(any_, gext(x2_v), _f32(0.0))
            by2 = jnp.where(any_, gext(y2_v), _f32(0.0))
            ssc = jnp.where(any_, s, _f32(0.0))
            ilbl = jnp.sum(jnp.where(ilm, lb_v[pl.ds(ibase, 16)], _i32(0)))
            slb = jnp.where(any_, ilbl, _i32(-1))
            kb = (k // 16) * 16
            kl = k - kb
            km = iota == kl
            ob1_v[pl.ds(kb, 16)] = jnp.where(km, bx1, ob1_v[pl.ds(kb, 16)])
            ob2_v[pl.ds(kb, 16)] = jnp.where(km, by1, ob2_v[pl.ds(kb, 16)])
            ob3_v[pl.ds(kb, 16)] = jnp.where(km, bx2, ob3_v[pl.ds(kb, 16)])
            ob4_v[pl.ds(kb, 16)] = jnp.where(km, by2, ob4_v[pl.ds(kb, 16)])
            obs_v[pl.ds(kb, 16)] = jnp.where(km, ssc, obs_v[pl.ds(kb, 16)])
            obl_v[pl.ds(kb, 16)] = jnp.where(km, slb, obl_v[pl.ds(kb, 16)])
            taken = any_.astype(_i32)
            a = a + jnp.where(take_h, taken, 0)
            b = b + jnp.where(take_h, 0, taken)
            return (a, b)

        lax.fori_loop(0, 30, mbody, (_i32(0), _i32(0)))

        outs = [
            pltpu.async_copy(src, dst, sem)
            for src, dst in ((ob1_v, ox1_h), (ob2_v, oy1_h), (ob3_v, ox2_h),
                             (ob4_v, oy2_h), (obs_v, osc_h), (obl_v, olb_h))
        ]
        for c in outs:
            c.wait()


def kernel(boxes, scores, labels):
    pad = _NPAD - _N
    x1 = jnp.pad(boxes[:, 0], (0, pad))
    y1 = jnp.pad(boxes[:, 1], (0, pad))
    x2 = jnp.pad(boxes[:, 2], (0, pad))
    y2 = jnp.pad(boxes[:, 3], (0, pad))
    sc = jnp.pad(scores, (0, pad), constant_values=-1.0)
    lb = jnp.pad(labels, (0, pad))
    bx1, by1, bx2, by2, osc, olb = _nms_sc(x1, y1, x2, y2, sc, lb)
    out_boxes = jnp.stack([bx1, by1, bx2, by2], axis=1)[:30]
    return out_boxes, osc[:30], olb[:30]
